# baseline (speedup 1.0000x reference)
import functools

import jax
import jax.numpy as jnp
from jax import lax
from jax.experimental import pallas as pl
from jax.experimental.pallas import tpu as pltpu
from jax.experimental.pallas import tpu_sc as plsc

D_MODEL = 1024
MAX_SEQ = 8192
SEP_ID = 102
S = 1024
WR = S + 16

B = 4
NB = MAX_SEQ // S
WPR = 8
CHUNK = MAX_SEQ // WPR
CPB = S // CHUNK


def _splat_sum(vec, red_v):
    idx = lax.iota(jnp.int32, 16)
    for k in (8, 4, 2, 1):
        red_v[...] = vec
        vec = vec + plsc.load_gather(red_v, [jnp.bitwise_xor(idx, k)])
    return vec


def _sc_meta_body(tok_hbm, bases_hbm, counts_hbm,
                  tok_v, allt_v, stage_v, red_v, bst_v, cst_v, totals_sh):
    c = lax.axis_index("c")
    s = lax.axis_index("s")
    row = 2 * c + s // WPR
    cid = s % WPR
    goff = row * MAX_SEQ + cid * CHUNK

    pltpu.sync_copy(tok_hbm.at[pl.ds(goff, CHUNK)], tok_v)

    ones16 = jnp.ones((16,), jnp.int32)
    zeros16 = jnp.zeros((16,), jnp.int32)

    acc = zeros16
    for i in range(CHUNK // 16):
        v = tok_v[pl.ds(i * 16, 16)]
        acc = acc + jnp.where(v == SEP_ID, ones16, zeros16)
    total = _splat_sum(acc, red_v)

    stage_v[...] = total
    pltpu.sync_copy(stage_v, totals_sh.at[pl.ds(s * 16, 16)])
    plsc.subcore_barrier()
    pltpu.sync_copy(totals_sh, allt_v)
    pref = zeros16
    srow0 = (s // WPR) * WPR
    for i in range(WPR):
        vr = allt_v[pl.ds((srow0 + i) * 16, 16)]
        iv = jnp.full((16,), i, jnp.int32)
        pref = pref + jnp.where(iv < cid, vr, zeros16)

    @pl.when(lax.rem(cid, CPB) == 0)
    def _emit():
        cntv = zeros16
        for t in range(CPB):
            cntv_t = allt_v[pl.ds((srow0 + cid + t) * 16, 16)]
            cntv = cntv + cntv_t
        bst_v[...] = pref
        cst_v[...] = cntv
        moff = (row * NB + cid // CPB) * 16
        pltpu.sync_copy(bst_v, bases_hbm.at[pl.ds(moff, 16)])
        pltpu.sync_copy(cst_v, counts_hbm.at[pl.ds(moff, 16)])


def _segment_meta(tokens):
    mesh = plsc.VectorSubcoreMesh(core_axis_name="c", subcore_axis_name="s")
    run = functools.partial(
        pl.kernel,
        out_type=[
            jax.ShapeDtypeStruct((B * NB * 16,), jnp.int32),
            jax.ShapeDtypeStruct((B * NB * 16,), jnp.int32),
        ],
        mesh=mesh,
        compiler_params=pltpu.CompilerParams(needs_layout_passes=False),
        scratch_types=[
            pltpu.VMEM((CHUNK,), jnp.int32),
            pltpu.VMEM((16 * 16,), jnp.int32),
            pltpu.VMEM((16,), jnp.int32),
            pltpu.VMEM((16,), jnp.int32),
            pltpu.VMEM((16,), jnp.int32),
            pltpu.VMEM((16,), jnp.int32),
            pltpu.VMEM_SHARED((16 * 16,), jnp.int32),
        ],
    )(_sc_meta_body)
    bases, counts = run(tokens.reshape(-1))
    return bases.reshape(B, NB, 1, 16), counts.reshape(B, NB, 1, 16)


def _main_kernel(meta_ref, x_ref, tok_ref, pe8_ref, peA_ref, peB_ref,
                 peany_ref, o_ref, wscr, dsem):
    b = pl.program_id(0)
    j = pl.program_id(1)
    base = meta_ref[0, b, j]
    cnt = meta_ref[1, b, j]
    w0 = meta_ref[2, b, j]
    p8i = meta_ref[3, b, j]
    xb = x_ref[0]

    @pl.when(cnt == 0)
    def _fast():
        off = jnp.minimum(base, MAX_SEQ - 1) - 8 * p8i
        sel = (jax.lax.broadcasted_iota(jnp.int32, (8, 1), 0)
               == off).astype(jnp.float32)
        row = jnp.sum(pe8_ref[...] * sel, axis=0, keepdims=True)
        o_ref[0] = xb + row

    def _pos_c():
        m = (tok_ref[0, 0] == SEP_ID).astype(jnp.int32)
        acc = m
        k = 1
        while k < S:
            acc = acc + jnp.concatenate(
                [jnp.zeros((1, k), jnp.int32), acc[:, :-k]], axis=1)
            k *= 2
        return jnp.minimum(base + acc, MAX_SEQ - 1)

    def _onehot_dot(w, r, nrows):
        oh = (jax.lax.broadcasted_iota(jnp.int32, (nrows, S), 0)
              == jnp.broadcast_to(r, (nrows, S))).astype(jnp.float32)
        return jax.lax.dot_general(oh, w, (((0,), (0,)), ((), ())),
                                   preferred_element_type=jnp.float32)

    @pl.when(jnp.logical_and(cnt > 0, cnt <= 32))
    def _small():
        r = _pos_c() - 32 * w0
        y = (_onehot_dot(peA_ref[...], r, 32)
             + _onehot_dot(peB_ref[...], r - 32, 32))
        o_ref[0] = xb + y

    @pl.when(cnt > 32)
    def _full():
        base8 = (jnp.minimum(base, MAX_SEQ - WR) // 8) * 8
        cp = pltpu.make_async_copy(
            peany_ref.at[pl.ds(base8, WR), :], wscr, dsem)
        cp.start()
        cp.wait()
        r = _pos_c() - base8
        o_ref[0] = xb + _onehot_dot(wscr[...], r, WR)


def _pos_kernel_tc(tok_ref, pos_ref):
    m = (tok_ref[...] == SEP_ID).astype(jnp.int32)
    acc = m
    k = 1
    while k < MAX_SEQ:
        zeros = jnp.zeros((acc.shape[0], k), jnp.int32)
        acc = acc + jnp.concatenate([zeros, acc[:, :-k]], axis=1)
        k *= 2
    pos_ref[...] = acc


def _segment_meta_tc(tokens):
    pos = pl.pallas_call(
        _pos_kernel_tc,
        out_shape=jax.ShapeDtypeStruct((B, MAX_SEQ), jnp.int32),
    )(tokens)
    ends = pos.reshape(B, NB, S)[:, :, -1]
    bases = jnp.concatenate(
        [jnp.zeros((B, 1), jnp.int32), ends[:, :-1]], axis=1)
    counts = ends - bases
    return bases, counts


@jax.jit
def kernel(x, tokens, pe):
    bases, counts = _segment_meta(tokens)
    bases = bases[:, :, 0, 0]
    counts = counts[:, :, 0, 0]
    tok4 = tokens.reshape(B, NB, 1, S)
    pe2 = pe[0]
    meta = jnp.stack([
        bases,
        counts,
        jnp.minimum(bases, MAX_SEQ - 64) // 32,
        jnp.minimum(bases, MAX_SEQ - 1) // 8,
    ])
    grid_spec = pltpu.PrefetchScalarGridSpec(
        num_scalar_prefetch=1,
        grid=(B, NB),
        in_specs=[
            pl.BlockSpec((1, S, D_MODEL), lambda b, j, m: (b, j, 0)),
            pl.BlockSpec((1, 1, 1, S), lambda b, j, m: (b, j, 0, 0)),
            pl.BlockSpec((8, D_MODEL), lambda b, j, m: (m[3, b, j], 0)),
            pl.BlockSpec((32, D_MODEL), lambda b, j, m: (m[2, b, j], 0)),
            pl.BlockSpec((32, D_MODEL), lambda b, j, m: (m[2, b, j] + 1, 0)),
            pl.BlockSpec(memory_space=pl.ANY),
        ],
        out_specs=pl.BlockSpec((1, S, D_MODEL), lambda b, j, m: (b, j, 0)),
        scratch_shapes=[
            pltpu.VMEM((WR, D_MODEL), jnp.float32),
            pltpu.SemaphoreType.DMA,
        ],
    )
    out = pl.pallas_call(
        _main_kernel,
        grid_spec=grid_spec,
        out_shape=jax.ShapeDtypeStruct((B, MAX_SEQ, D_MODEL), jnp.float32),
    )(meta, x, tok4, pe2, pe2, pe2, pe2)
    return out


# speedup vs baseline: 1.0134x; 1.0134x over previous
import functools

import jax
import jax.numpy as jnp
from jax import lax
from jax.experimental import pallas as pl
from jax.experimental.pallas import tpu as pltpu
from jax.experimental.pallas import tpu_sc as plsc

D_MODEL = 1024
MAX_SEQ = 8192
SEP_ID = 102
S = 1024
WR = S + 16

B = 4
NB = MAX_SEQ // S
SC_CORES = 1
WPR = 4
CHUNK = MAX_SEQ // WPR
KB = CHUNK // S


def _splat_sum(vec, red_v):
    idx = lax.iota(jnp.int32, 16)
    for k in (8, 4, 2, 1):
        red_v[...] = vec
        vec = vec + plsc.load_gather(red_v, [jnp.bitwise_xor(idx, k)])
    return vec


def _sc_meta_body(tok_hbm, bases_hbm, counts_hbm,
                  tok_v, allt_v, stage_v, red_v, bst_v, cst_v, totals_sh):
    s = lax.axis_index("s")
    row = s // WPR
    cid = s % WPR
    goff = row * MAX_SEQ + cid * CHUNK

    pltpu.sync_copy(tok_hbm.at[pl.ds(goff, CHUNK)], tok_v)

    ones16 = jnp.ones((16,), jnp.int32)
    zeros16 = jnp.zeros((16,), jnp.int32)

    bsum = []
    for k in range(KB):
        acc = zeros16
        for i in range(S // 16):
            v = tok_v[pl.ds((k * (S // 16) + i) * 16, 16)]
            acc = acc + jnp.where(v == SEP_ID, ones16, zeros16)
        bsum.append(_splat_sum(acc, red_v))
    total = bsum[0]
    for k in range(1, KB):
        total = total + bsum[k]

    stage_v[...] = total
    pltpu.sync_copy(stage_v, totals_sh.at[pl.ds(s * 16, 16)])
    plsc.subcore_barrier()
    pltpu.sync_copy(totals_sh, allt_v)
    pref = zeros16
    srow0 = (s // WPR) * WPR
    for i in range(WPR):
        vr = allt_v[pl.ds((srow0 + i) * 16, 16)]
        iv = jnp.full((16,), i, jnp.int32)
        pref = pref + jnp.where(iv < cid, vr, zeros16)

    prev = pref
    for k in range(KB):
        bst_v[pl.ds(k * 16, 16)] = prev
        cst_v[pl.ds(k * 16, 16)] = bsum[k]
        prev = prev + bsum[k]
    moff = (row * NB + cid * KB) * 16
    pltpu.sync_copy(bst_v, bases_hbm.at[pl.ds(moff, KB * 16)])
    pltpu.sync_copy(cst_v, counts_hbm.at[pl.ds(moff, KB * 16)])


def _segment_meta(tokens):
    mesh = plsc.VectorSubcoreMesh(core_axis_name="c", subcore_axis_name="s",
                                  num_cores=SC_CORES)
    run = functools.partial(
        pl.kernel,
        out_type=[
            jax.ShapeDtypeStruct((B * NB * 16,), jnp.int32),
            jax.ShapeDtypeStruct((B * NB * 16,), jnp.int32),
        ],
        mesh=mesh,
        compiler_params=pltpu.CompilerParams(needs_layout_passes=False),
        scratch_types=[
            pltpu.VMEM((CHUNK,), jnp.int32),
            pltpu.VMEM((16 * 16,), jnp.int32),
            pltpu.VMEM((16,), jnp.int32),
            pltpu.VMEM((16,), jnp.int32),
            pltpu.VMEM((KB * 16,), jnp.int32),
            pltpu.VMEM((KB * 16,), jnp.int32),
            pltpu.VMEM_SHARED((16 * 16,), jnp.int32),
        ],
    )(_sc_meta_body)
    bases, counts = run(tokens.reshape(-1))
    return bases.reshape(B, NB, 1, 16), counts.reshape(B, NB, 1, 16)


def _main_kernel(meta_ref, x_ref, tok_ref, pe8_ref, peA_ref, peB_ref,
                 peany_ref, o_ref, wscr, dsem):
    b = pl.program_id(0)
    j = pl.program_id(1)
    base = meta_ref[0, b, j]
    cnt = meta_ref[1, b, j]
    w0 = meta_ref[2, b, j]
    p8i = meta_ref[3, b, j]
    xb = x_ref[0]

    @pl.when(cnt == 0)
    def _fast():
        off = jnp.minimum(base, MAX_SEQ - 1) - 8 * p8i
        sel = (jax.lax.broadcasted_iota(jnp.int32, (8, 1), 0)
               == off).astype(jnp.float32)
        row = jnp.sum(pe8_ref[...] * sel, axis=0, keepdims=True)
        o_ref[0] = xb + row

    def _pos_c():
        m = (tok_ref[0, 0] == SEP_ID).astype(jnp.int32)
        acc = m
        k = 1
        while k < S:
            acc = acc + jnp.concatenate(
                [jnp.zeros((1, k), jnp.int32), acc[:, :-k]], axis=1)
            k *= 2
        return jnp.minimum(base + acc, MAX_SEQ - 1)

    def _onehot_dot(w, r, nrows):
        oh = (jax.lax.broadcasted_iota(jnp.int32, (nrows, S), 0)
              == jnp.broadcast_to(r, (nrows, S))).astype(jnp.float32)
        return jax.lax.dot_general(oh, w, (((0,), (0,)), ((), ())),
                                   preferred_element_type=jnp.float32)

    @pl.when(jnp.logical_and(cnt > 0, cnt <= 32))
    def _small():
        r = _pos_c() - 32 * w0
        y = (_onehot_dot(peA_ref[...], r, 32)
             + _onehot_dot(peB_ref[...], r - 32, 32))
        o_ref[0] = xb + y

    @pl.when(cnt > 32)
    def _full():
        base8 = (jnp.minimum(base, MAX_SEQ - WR) // 8) * 8
        cp = pltpu.make_async_copy(
            peany_ref.at[pl.ds(base8, WR), :], wscr, dsem)
        cp.start()
        cp.wait()
        r = _pos_c() - base8
        o_ref[0] = xb + _onehot_dot(wscr[...], r, WR)


def _pos_kernel_tc(tok_ref, pos_ref):
    m = (tok_ref[...] == SEP_ID).astype(jnp.int32)
    acc = m
    k = 1
    while k < MAX_SEQ:
        zeros = jnp.zeros((acc.shape[0], k), jnp.int32)
        acc = acc + jnp.concatenate([zeros, acc[:, :-k]], axis=1)
        k *= 2
    pos_ref[...] = acc


def _segment_meta_tc(tokens):
    pos = pl.pallas_call(
        _pos_kernel_tc,
        out_shape=jax.ShapeDtypeStruct((B, MAX_SEQ), jnp.int32),
    )(tokens)
    ends = pos.reshape(B, NB, S)[:, :, -1]
    bases = jnp.concatenate(
        [jnp.zeros((B, 1), jnp.int32), ends[:, :-1]], axis=1)
    counts = ends - bases
    return bases, counts


@jax.jit
def kernel(x, tokens, pe):
    bases, counts = _segment_meta(tokens)
    bases = bases[:, :, 0, 0]
    counts = counts[:, :, 0, 0]
    tok4 = tokens.reshape(B, NB, 1, S)
    pe2 = pe[0]
    meta = jnp.stack([
        bases,
        counts,
        jnp.minimum(bases, MAX_SEQ - 64) // 32,
        jnp.minimum(bases, MAX_SEQ - 1) // 8,
    ])
    grid_spec = pltpu.PrefetchScalarGridSpec(
        num_scalar_prefetch=1,
        grid=(B, NB),
        in_specs=[
            pl.BlockSpec((1, S, D_MODEL), lambda b, j, m: (b, j, 0)),
            pl.BlockSpec((1, 1, 1, S), lambda b, j, m: (b, j, 0, 0)),
            pl.BlockSpec((8, D_MODEL), lambda b, j, m: (m[3, b, j], 0)),
            pl.BlockSpec((32, D_MODEL), lambda b, j, m: (m[2, b, j], 0)),
            pl.BlockSpec((32, D_MODEL), lambda b, j, m: (m[2, b, j] + 1, 0)),
            pl.BlockSpec(memory_space=pl.ANY),
        ],
        out_specs=pl.BlockSpec((1, S, D_MODEL), lambda b, j, m: (b, j, 0)),
        scratch_shapes=[
            pltpu.VMEM((WR, D_MODEL), jnp.float32),
            pltpu.SemaphoreType.DMA,
        ],
    )
    out = pl.pallas_call(
        _main_kernel,
        grid_spec=grid_spec,
        out_shape=jax.ShapeDtypeStruct((B, MAX_SEQ, D_MODEL), jnp.float32),
    )(meta, x, tok4, pe2, pe2, pe2, pe2)
    return out
